# baseline (speedup 1.0000x reference)
import math

import numpy as np
import jax
import jax.numpy as jnp
from jax.experimental import pallas as pl

D = 128
_div = np.exp(-math.log(10000.0) * np.arange(0, D, 2, dtype=np.float32) / D)
_divfull = np.repeat(_div, 2).astype(np.float32)
_evenmask = (np.arange(D) % 2 == 0).astype(np.int32)

BLK = 1024


def _body(t_ref, div_ref, even_ref, o_ref):
    pos = t_ref[...].astype(jnp.float32)
    x = pos * div_ref[...]
    o_ref[...] = jnp.where(even_ref[...] != 0, jnp.sin(x), jnp.cos(x))


def kernel(t, pos_encoding):
    B = t.shape[0]
    divfull = jnp.asarray(_divfull)[None, :]
    even = jnp.asarray(_evenmask)[None, :]
    f = pl.pallas_call(
        _body,
        grid=(B // BLK,),
        in_specs=[pl.BlockSpec((BLK, 1), lambda i: (i, 0)),
                  pl.BlockSpec((1, D), lambda i: (0, 0)),
                  pl.BlockSpec((1, D), lambda i: (0, 0))],
        out_specs=pl.BlockSpec((BLK, D), lambda i: (i, 0)),
        out_shape=jax.ShapeDtypeStruct((B, D), jnp.float32),
    )
    return f(t, divfull, even)


# speedup vs baseline: 1.0604x; 1.0604x over previous
import functools
import math

import numpy as np
import jax
import jax.numpy as jnp
from jax import lax
from jax.experimental import pallas as pl
from jax.experimental.pallas import tpu as pltpu
from jax.experimental.pallas import tpu_sc as plsc

NC = 2
NS = 16
NW = NC * NS
CHUNK = 128

D = 128
_div = np.exp(-math.log(10000.0) * np.arange(0, D, 2, dtype=np.float32) / D)
_divfull = np.repeat(_div, 2).astype(np.float32)
_evenmask = (np.arange(D) % 2 == 0).astype(np.float32)

_C1 = np.float32(3.140625)
_rem = math.pi - float(_C1)
_e = math.floor(math.log2(abs(_rem)))
_C2 = np.float32(round(_rem / 2.0 ** (_e - 8)) * 2.0 ** (_e - 8))
_C3 = np.float32(math.pi - float(_C1) - float(_C2))
_INV_PI = np.float32(1.0 / math.pi)

BLK = 1024

B_SC = 8192


@functools.lru_cache(maxsize=None)
def _make_sc_gather(B, V):
    b_per_w = B // NW
    K = b_per_w // CHUNK
    mesh = plsc.VectorSubcoreMesh(core_axis_name="c", subcore_axis_name="s")

    @functools.partial(
        pl.kernel,
        mesh=mesh,
        out_type=jax.ShapeDtypeStruct((B, D), jnp.float32),
        scratch_types=[
            pltpu.VMEM((K, CHUNK), jnp.int32),
            pltpu.VMEM((b_per_w, D), jnp.float32),
            pltpu.SemaphoreType.DMA,
        ],
    )
    def k(idx_hbm, table_hbm, out_hbm, idx_v, rows_v, gsem):
        wid = lax.axis_index("s") * NC + lax.axis_index("c")
        pltpu.sync_copy(idx_hbm.at[wid], idx_v)
        gathers = [
            pltpu.async_copy(
                table_hbm.at[idx_v.at[j]],
                rows_v.at[pl.ds(j * CHUNK, CHUNK)],
                gsem,
            )
            for j in range(K)
        ]
        for c in gathers:
            c.wait()
        pltpu.sync_copy(rows_v, out_hbm.at[pl.ds(wid * b_per_w, b_per_w)])

    return k


def _tc_body(t_ref, div_ref, even_ref, o_ref):
    pos = t_ref[...].astype(jnp.float32)
    x = pos * div_ref[...]
    n = jnp.round(x * _INV_PI)
    y = ((x - n * _C1) - n * _C2) - n * _C3
    y2 = y * y
    sp = y * (1.0 + y2 * (np.float32(-1.0 / 6) + y2 * (np.float32(1.0 / 120)
         + y2 * (np.float32(-1.0 / 5040) + y2 * np.float32(1.0 / 362880)))))
    cp = 1.0 + y2 * (np.float32(-0.5) + y2 * (np.float32(1.0 / 24)
         + y2 * (np.float32(-1.0 / 720) + y2 * (np.float32(1.0 / 40320)
         + y2 * np.float32(-1.0 / 3628800)))))
    half = n * 0.5
    sign = 1.0 - 4.0 * (half - jnp.floor(half))
    o_ref[...] = jnp.where(even_ref[...] != 0, sp, cp) * sign


def _tc_sincos(t):
    B = t.shape[0]
    divfull = jnp.asarray(_divfull)[None, :]
    even = jnp.asarray(_evenmask)[None, :]
    f = pl.pallas_call(
        _tc_body,
        grid=(B // BLK,),
        in_specs=[pl.BlockSpec((BLK, 1), lambda i: (i, 0)),
                  pl.BlockSpec((1, D), lambda i: (0, 0)),
                  pl.BlockSpec((1, D), lambda i: (0, 0))],
        out_specs=pl.BlockSpec((BLK, D), lambda i: (i, 0)),
        out_shape=jax.ShapeDtypeStruct((B, D), jnp.float32),
    )
    return f(t, divfull, even)


def kernel(t, pos_encoding):
    B = t.shape[0]
    V, _ = pos_encoding.shape
    idx = t[:B_SC].reshape(NW, B_SC // (NW * CHUNK), CHUNK).astype(jnp.int32)
    out_sc = _make_sc_gather(B_SC, V)(idx, pos_encoding)
    out_tc = _tc_sincos(t[B_SC:])
    return jnp.concatenate([out_sc, out_tc], axis=0)


# speedup vs baseline: 1.4922x; 1.4072x over previous
import functools

import jax
import jax.numpy as jnp
from jax import lax
from jax.experimental import pallas as pl
from jax.experimental.pallas import tpu as pltpu
from jax.experimental.pallas import tpu_sc as plsc

NC = 2
NS = 16
NW = NC * NS
CHUNK = 128


@functools.lru_cache(maxsize=None)
def _make_gather(B, V, D):
    b_per_w = B // NW
    K = b_per_w // CHUNK
    mesh = plsc.VectorSubcoreMesh(core_axis_name="c", subcore_axis_name="s")

    @functools.partial(
        pl.kernel,
        mesh=mesh,
        out_type=jax.ShapeDtypeStruct((B, D), jnp.float32),
        scratch_types=[
            pltpu.VMEM((K, CHUNK), jnp.int32),
            pltpu.VMEM((b_per_w, D), jnp.float32),
            pltpu.SemaphoreType.DMA,
        ],
    )
    def k(idx_hbm, table_hbm, out_hbm, idx_v, rows_v, gsem):
        wid = lax.axis_index("s") * NC + lax.axis_index("c")
        pltpu.sync_copy(idx_hbm.at[wid], idx_v)
        gathers = [
            pltpu.async_copy(
                table_hbm.at[idx_v.at[j]],
                rows_v.at[pl.ds(j * CHUNK, CHUNK)],
                gsem,
            )
            for j in range(K)
        ]
        for c in gathers:
            c.wait()
        pltpu.sync_copy(rows_v, out_hbm.at[pl.ds(wid * b_per_w, b_per_w)])

    return k


def kernel(t, pos_encoding):
    B = t.shape[0]
    V, D = pos_encoding.shape
    idx = t.reshape(NW, B // (NW * CHUNK), CHUNK).astype(jnp.int32)
    return _make_gather(B, V, D)(idx, pos_encoding)
